# baseline (speedup 1.0000x reference)
import jax
import jax.numpy as jnp
from jax import lax
from jax.experimental import pallas as pl
from jax.experimental.pallas import tpu as pltpu
from jax.experimental.pallas import tpu_sc as plsc

N = 10000
E = 160000
F_IN = 256
F_HID = 256
F_OUT = 64
FH = F_HID // 2
NC = 2
NS = 16
NP = 10240
ROWS_T = NP // NS

CH = 200
EPT = 2 * E // (NC * NS)

MB = 1000

_mesh = lambda: plsc.VectorSubcoreMesh(
    core_axis_name="c", subcore_axis_name="s", num_cores=NC, num_subcores=NS)


def _fill(ref, nrows, ncols, value):
    v = jnp.full((16,), value, jnp.float32)
    per_row = ncols // 16

    def body(k, carry):
        ref[k // per_row, pl.ds((k % per_row) * 16, 16)] = v
        return carry

    lax.fori_loop(0, nrows * per_row, body, 0)


def _agg_body(y_hbm, src_hbm, dst_hbm, out_hbm, srcv, dstv, rows_v, table_s):
    c = lax.axis_index("c")
    s = lax.axis_index("s")
    _fill(rows_v, CH, 128, 0.0)
    for off, sz in ((0, 200), (200, 200), (400, 200), (600, 40)):
        pltpu.sync_copy(rows_v.at[pl.ds(0, sz)],
                        table_s.at[pl.ds(s * ROWS_T + off, sz)])
    plsc.subcore_barrier()

    base = (c * NS + s) * EPT

    def chunk(k, carry):
        eb = pl.multiple_of(base + k * CH, 8)
        pltpu.sync_copy(src_hbm.at[pl.ds(eb, CH)], srcv)
        pltpu.sync_copy(dst_hbm.at[pl.ds(eb, CH)], dstv)
        pltpu.sync_copy(y_hbm.at[srcv], rows_v)
        pltpu.sync_copy(rows_v, table_s.at[dstv], add=True)
        return carry

    lax.fori_loop(0, EPT // CH, chunk, 0)
    plsc.subcore_barrier()
    ro = pl.multiple_of(c * NP + s * ROWS_T, 8)
    for off, sz in ((0, 200), (200, 200), (400, 200), (600, 40)):
        pltpu.sync_copy(table_s.at[pl.ds(s * ROWS_T + off, sz)],
                        rows_v.at[pl.ds(0, sz)])
        pltpu.sync_copy(rows_v.at[pl.ds(0, sz)],
                        out_hbm.at[pl.ds(ro + off, sz)])


def _edge_agg(y_flat, src2, dst2):
    k = pl.kernel(
        _agg_body,
        out_type=jax.ShapeDtypeStruct((NC * NP, 128), jnp.float32),
        mesh=_mesh(),
        scratch_types=[
            pltpu.VMEM((CH,), jnp.int32),
            pltpu.VMEM((CH,), jnp.int32),
            pltpu.VMEM((CH, 128), jnp.float32),
            pltpu.VMEM_SHARED((NP, 128), jnp.float32),
        ],
    )
    return k(y_flat, src2, dst2)


def _dis_block(dp_ref):
    return lax.rsqrt(dp_ref[0, :, 0:1] + 1.0)


def _tca_body(x_ref, w_ref, dp_ref, y_ref):
    dis = _dis_block(dp_ref)
    y_ref[...] = jnp.dot(x_ref[...], w_ref[...],
                         preferred_element_type=jnp.float32) * dis


def _tc_a(x, W1, degp):
    return pl.pallas_call(
        _tca_body,
        grid=(N // MB, NC),
        in_specs=[
            pl.BlockSpec((MB, F_IN), lambda i, j: (i, 0)),
            pl.BlockSpec((F_IN, FH), lambda i, j: (0, j)),
            pl.BlockSpec((1, MB, 128), lambda i, j: (0, i, 0)),
        ],
        out_specs=pl.BlockSpec((MB, FH), lambda i, j: (j * (N // MB) + i, 0)),
        out_shape=jax.ShapeDtypeStruct((NC * N, FH), jnp.float32),
    )(x, W1, degp)


def _tcb_body(a_ref, y_ref, dp_ref, w_ref, b_ref, o_ref):
    dis = _dis_block(dp_ref)
    agg = jnp.concatenate([a_ref[0], a_ref[1]], axis=1)
    y = jnp.concatenate([y_ref[0], y_ref[1]], axis=1)
    h = jnp.maximum((agg + y) * dis + b_ref[...], 0.0)
    y2 = jnp.dot(h, w_ref[...], preferred_element_type=jnp.float32) * dis
    o_ref[...] = jnp.concatenate(
        [y2, jnp.zeros((MB, 128 - F_OUT), jnp.float32)], axis=1)


def _tc_b(agg1, y1, degp, W2, b1):
    return pl.pallas_call(
        _tcb_body,
        grid=(N // MB,),
        in_specs=[
            pl.BlockSpec((NC, MB, FH), lambda i: (0, i, 0)),
            pl.BlockSpec((NC, MB, FH), lambda i: (0, i, 0)),
            pl.BlockSpec((1, MB, 128), lambda i: (0, i, 0)),
            pl.BlockSpec((F_HID, F_OUT), lambda i: (0, 0)),
            pl.BlockSpec((1, F_HID), lambda i: (0, 0)),
        ],
        out_specs=pl.BlockSpec((MB, 128), lambda i: (i, 0)),
        out_shape=jax.ShapeDtypeStruct((NC * N, 128), jnp.float32),
    )(agg1, y1, degp, W2, b1)


def _tcc_body(a_ref, y_ref, dp_ref, b_ref, o_ref):
    dis = _dis_block(dp_ref)
    o_ref[...] = ((a_ref[0, :, :F_OUT] + y_ref[:, :F_OUT]) * dis
                  + b_ref[...])


def _tc_c(agg2, y2p, degp, b2):
    return pl.pallas_call(
        _tcc_body,
        grid=(N // MB,),
        in_specs=[
            pl.BlockSpec((1, MB, 128), lambda i: (0, i, 0)),
            pl.BlockSpec((MB, 128), lambda i: (i, 0)),
            pl.BlockSpec((1, MB, 128), lambda i: (0, i, 0)),
            pl.BlockSpec((1, F_OUT), lambda i: (0, 0)),
        ],
        out_specs=pl.BlockSpec((MB, F_OUT), lambda i: (i, 0)),
        out_shape=jax.ShapeDtypeStruct((N, F_OUT), jnp.float32),
    )(agg2, y2p, degp, b2)


def kernel(x, edge_index, W1, b1, W2, b2):
    src = edge_index[0].astype(jnp.int32)
    dst = edge_index[1].astype(jnp.int32)
    dst2 = jnp.concatenate([dst, dst])

    ones_tab = jnp.broadcast_to(x[:1, :1] * 0.0 + 1.0, (NC * N, 128))
    zidx = dst2 * 0
    degp = _edge_agg(ones_tab, zidx, dst2).reshape(NC, NP, 128)

    y1 = _tc_a(x, W1, degp)
    src_l1 = jnp.concatenate([src, src + N])
    agg1 = _edge_agg(y1, src_l1, dst2).reshape(NC, NP, 128)

    y2p = _tc_b(agg1, y1.reshape(NC, N, FH), degp, W2, b1.reshape(1, F_HID))
    src_l2 = jnp.concatenate([src, src])
    agg2 = _edge_agg(y2p, src_l2, dst2).reshape(NC, NP, 128)

    return _tc_c(agg2, y2p, degp, b2.reshape(1, F_OUT))


# speedup vs baseline: 18.3256x; 18.3256x over previous
import jax
import jax.numpy as jnp
from jax import lax
from jax.experimental import pallas as pl
from jax.experimental.pallas import tpu as pltpu
from jax.experimental.pallas import tpu_sc as plsc

N = 10000
E = 160000
F_IN = 256
F_HID = 256
F_OUT = 64
FH = F_HID // 2
NC = 2
NS = 16
NP = 10240
ROWS_T = NP // NS

CH = 200
EPT = 2 * E // (NC * NS)

MB = 1000

_mesh = lambda: plsc.VectorSubcoreMesh(
    core_axis_name="c", subcore_axis_name="s", num_cores=NC, num_subcores=NS)


def _fill(ref, nrows, ncols, value):
    v = jnp.full((16,), value, jnp.float32)
    per_row = ncols // 16

    def body(k, carry):
        ref[k // per_row, pl.ds((k % per_row) * 16, 16)] = v
        return carry

    lax.fori_loop(0, nrows * per_row, body, 0)


def _agg_body(y_hbm, src_hbm, dst_hbm, out_hbm, srcv, dstv, rows_v, table_s):
    c = lax.axis_index("c")
    s = lax.axis_index("s")
    _fill(rows_v, CH, 128, 0.0)
    for off, sz in ((0, 200), (200, 200), (400, 200), (600, 40)):
        pltpu.sync_copy(rows_v.at[pl.ds(0, sz)],
                        table_s.at[pl.ds(s * ROWS_T + off, sz)])
    plsc.subcore_barrier()

    base = (c * NS + s) * EPT

    def chunk(k, carry):
        eb = pl.multiple_of(base + k * CH, 8)
        pltpu.sync_copy(src_hbm.at[pl.ds(eb, CH)], srcv)
        pltpu.sync_copy(dst_hbm.at[pl.ds(eb, CH)], dstv)
        pltpu.sync_copy(y_hbm.at[srcv], rows_v)
        pltpu.sync_copy(rows_v, table_s.at[dstv], add=True)
        return carry

    lax.fori_loop(0, EPT // CH, chunk, 0)
    plsc.subcore_barrier()
    ro = pl.multiple_of(c * NP + s * ROWS_T, 8)
    for off, sz in ((0, 200), (200, 200), (400, 200), (600, 40)):
        pltpu.sync_copy(table_s.at[pl.ds(s * ROWS_T + off, sz)],
                        rows_v.at[pl.ds(0, sz)])
        pltpu.sync_copy(rows_v.at[pl.ds(0, sz)],
                        out_hbm.at[pl.ds(ro + off, sz)])


def _edge_agg(y_flat, src2, dst2):
    k = pl.kernel(
        _agg_body,
        out_type=jax.ShapeDtypeStruct((NC * NP, 128), jnp.float32),
        mesh=_mesh(),
        scratch_types=[
            pltpu.VMEM((CH,), jnp.int32),
            pltpu.VMEM((CH,), jnp.int32),
            pltpu.VMEM((CH, 128), jnp.float32),
            pltpu.VMEM_SHARED((NP, 128), jnp.float32),
        ],
    )
    return k(y_flat, src2, dst2)


def _dis_block(dp_ref):
    return lax.rsqrt(dp_ref[0, :, 0:1] + 1.0)


def _tca_body(x_ref, w_ref, dp_ref, y_ref):
    dis = _dis_block(dp_ref)
    y_ref[...] = jnp.dot(x_ref[...], w_ref[...],
                         preferred_element_type=jnp.float32) * dis


def _tc_a(x, W1, degp):
    return pl.pallas_call(
        _tca_body,
        grid=(N // MB, NC),
        in_specs=[
            pl.BlockSpec((MB, F_IN), lambda i, j: (i, 0)),
            pl.BlockSpec((F_IN, FH), lambda i, j: (0, j)),
            pl.BlockSpec((1, MB, 128), lambda i, j: (0, i, 0)),
        ],
        out_specs=pl.BlockSpec((MB, FH), lambda i, j: (j * (N // MB) + i, 0)),
        out_shape=jax.ShapeDtypeStruct((NC * N, FH), jnp.float32),
    )(x, W1, degp)


def _tcb_body(a_ref, y_ref, dp_ref, w_ref, b_ref, o_ref):
    dis = _dis_block(dp_ref)
    agg = jnp.concatenate([a_ref[0], a_ref[1]], axis=1)
    y = jnp.concatenate([y_ref[0], y_ref[1]], axis=1)
    h = jnp.maximum((agg + y) * dis + b_ref[...], 0.0)
    y2 = jnp.dot(h, w_ref[...], preferred_element_type=jnp.float32) * dis
    o_ref[...] = jnp.concatenate(
        [y2, jnp.zeros((MB, 128 - F_OUT), jnp.float32)], axis=1)


def _tc_b(agg1, y1, degp, W2, b1):
    return pl.pallas_call(
        _tcb_body,
        grid=(N // MB,),
        in_specs=[
            pl.BlockSpec((NC, MB, FH), lambda i: (0, i, 0)),
            pl.BlockSpec((NC, MB, FH), lambda i: (0, i, 0)),
            pl.BlockSpec((1, MB, 128), lambda i: (0, i, 0)),
            pl.BlockSpec((F_HID, F_OUT), lambda i: (0, 0)),
            pl.BlockSpec((1, F_HID), lambda i: (0, 0)),
        ],
        out_specs=pl.BlockSpec((MB, 128), lambda i: (i, 0)),
        out_shape=jax.ShapeDtypeStruct((NC * N, 128), jnp.float32),
    )(agg1, y1, degp, W2, b1)


def _tcc_body(a_ref, y_ref, dp_ref, b_ref, o_ref):
    dis = _dis_block(dp_ref)
    o_ref[...] = ((a_ref[0, :, :F_OUT] + y_ref[:, :F_OUT]) * dis
                  + b_ref[...])


def _tc_c(agg2, y2p, degp, b2):
    return pl.pallas_call(
        _tcc_body,
        grid=(N // MB,),
        in_specs=[
            pl.BlockSpec((1, MB, 128), lambda i: (0, i, 0)),
            pl.BlockSpec((MB, 128), lambda i: (i, 0)),
            pl.BlockSpec((1, MB, 128), lambda i: (0, i, 0)),
            pl.BlockSpec((1, F_OUT), lambda i: (0, 0)),
        ],
        out_specs=pl.BlockSpec((MB, F_OUT), lambda i: (i, 0)),
        out_shape=jax.ShapeDtypeStruct((N, F_OUT), jnp.float32),
    )(agg2, y2p, degp, b2)


def kernel(x, edge_index, W1, b1, W2, b2):
    src = edge_index[0].astype(jnp.int32)
    dst = edge_index[1].astype(jnp.int32)
    dst2 = jnp.concatenate([dst, dst])

    src_l1 = jnp.concatenate([src, src + N])
    ones_tab = jnp.broadcast_to(x[:1, :1] * 0.0 + 1.0, (NC * N, 128))
    degp = _edge_agg(ones_tab, src_l1, dst2).reshape(NC, NP, 128)

    y1 = _tc_a(x, W1, degp)
    agg1 = _edge_agg(y1, src_l1, dst2).reshape(NC, NP, 128)

    y2p = _tc_b(agg1, y1.reshape(NC, N, FH), degp, W2, b1.reshape(1, F_HID))
    src_l2 = jnp.concatenate([src, src])
    agg2 = _edge_agg(y2p, src_l2, dst2).reshape(NC, NP, 128)

    return _tc_c(agg2, y2p, degp, b2.reshape(1, F_OUT))


# speedup vs baseline: 22.6913x; 1.2382x over previous
import jax
import jax.numpy as jnp
from jax import lax
from jax.experimental import pallas as pl
from jax.experimental.pallas import tpu as pltpu
from jax.experimental.pallas import tpu_sc as plsc

N = 10000
E = 160000
F_IN = 256
F_HID = 256
F_OUT = 64
FH = F_HID // 2
NC = 2
NS = 16
NP = 10240
ROWS_T = NP // NS

CH = 200
EPT = 2 * E // (NC * NS)

MB = 1000

_mesh = lambda: plsc.VectorSubcoreMesh(
    core_axis_name="c", subcore_axis_name="s", num_cores=NC, num_subcores=NS)


def _fill(ref, nrows, ncols, value):
    v = jnp.full((16,), value, jnp.float32)
    per_row = ncols // 16

    def body(k, carry):
        ref[k // per_row, pl.ds((k % per_row) * 16, 16)] = v
        return carry

    lax.fori_loop(0, nrows * per_row, body, 0)


NCHUNK = EPT // CH


def _agg_body(y_hbm, src_hbm, dst_hbm, out_hbm,
              srcva, dstva, srcvb, dstvb, rows_v, table_s, sema, semb):
    c = lax.axis_index("c")
    s = lax.axis_index("s")
    _fill(rows_v, CH, 128, 0.0)
    for off, sz in ((0, 200), (200, 200), (400, 200), (600, 40)):
        pltpu.sync_copy(rows_v.at[pl.ds(0, sz)],
                        table_s.at[pl.ds(s * ROWS_T + off, sz)])
    plsc.subcore_barrier()

    base = (c * NS + s) * EPT

    def drain(buf, sem):
        pltpu.make_async_copy(src_hbm.at[pl.ds(0, CH)], buf, sem).wait()

    def issue(eb, sbuf, dbuf, sem):
        pltpu.async_copy(src_hbm.at[pl.ds(eb, CH)], sbuf, sem)
        pltpu.async_copy(dst_hbm.at[pl.ds(eb, CH)], dbuf, sem)

    def work(sbuf, dbuf):
        pltpu.sync_copy(y_hbm.at[sbuf], rows_v)
        pltpu.sync_copy(rows_v, table_s.at[dbuf], add=True)

    issue(pl.multiple_of(base, 8), srcva, dstva, sema)

    def pair(m, carry):
        eb_b = pl.multiple_of(base + (2 * m + 1) * CH, 8)
        eb_n = pl.multiple_of(base + ((2 * m + 2) % NCHUNK) * CH, 8)
        drain(srcva, sema)
        drain(dstva, sema)
        issue(eb_b, srcvb, dstvb, semb)
        work(srcva, dstva)
        drain(srcvb, semb)
        drain(dstvb, semb)
        issue(eb_n, srcva, dstva, sema)
        work(srcvb, dstvb)
        return carry

    lax.fori_loop(0, NCHUNK // 2, pair, 0)
    drain(srcva, sema)
    drain(dstva, sema)
    plsc.subcore_barrier()
    ro = pl.multiple_of(c * NP + s * ROWS_T, 8)
    for off, sz in ((0, 200), (200, 200), (400, 200), (600, 40)):
        pltpu.sync_copy(table_s.at[pl.ds(s * ROWS_T + off, sz)],
                        rows_v.at[pl.ds(0, sz)])
        pltpu.sync_copy(rows_v.at[pl.ds(0, sz)],
                        out_hbm.at[pl.ds(ro + off, sz)])


def _edge_agg(y_flat, src2, dst2):
    k = pl.kernel(
        _agg_body,
        out_type=jax.ShapeDtypeStruct((NC * NP, 128), jnp.float32),
        mesh=_mesh(),
        scratch_types=[
            pltpu.VMEM((CH,), jnp.int32),
            pltpu.VMEM((CH,), jnp.int32),
            pltpu.VMEM((CH,), jnp.int32),
            pltpu.VMEM((CH,), jnp.int32),
            pltpu.VMEM((CH, 128), jnp.float32),
            pltpu.VMEM_SHARED((NP, 128), jnp.float32),
            pltpu.SemaphoreType.DMA,
            pltpu.SemaphoreType.DMA,
        ],
    )
    return k(y_flat, src2, dst2)


def _dis_block(dp_ref):
    return lax.rsqrt(dp_ref[0, :, 0:1] + 1.0)


def _tca_body(x_ref, w_ref, dp_ref, y_ref):
    dis = _dis_block(dp_ref)
    y_ref[...] = jnp.dot(x_ref[...], w_ref[...],
                         preferred_element_type=jnp.float32) * dis


def _tc_a(x, W1, degp):
    return pl.pallas_call(
        _tca_body,
        grid=(N // MB, NC),
        in_specs=[
            pl.BlockSpec((MB, F_IN), lambda i, j: (i, 0)),
            pl.BlockSpec((F_IN, FH), lambda i, j: (0, j)),
            pl.BlockSpec((1, MB, 128), lambda i, j: (0, i, 0)),
        ],
        out_specs=pl.BlockSpec((MB, FH), lambda i, j: (j * (N // MB) + i, 0)),
        out_shape=jax.ShapeDtypeStruct((NC * N, FH), jnp.float32),
    )(x, W1, degp)


def _tcb_body(a_ref, y_ref, dp_ref, w_ref, b_ref, o_ref):
    dis = _dis_block(dp_ref)
    agg = jnp.concatenate([a_ref[0], a_ref[1]], axis=1)
    y = jnp.concatenate([y_ref[0], y_ref[1]], axis=1)
    h = jnp.maximum((agg + y) * dis + b_ref[...], 0.0)
    y2 = jnp.dot(h, w_ref[...], preferred_element_type=jnp.float32) * dis
    o_ref[...] = jnp.concatenate(
        [y2, jnp.zeros((MB, 128 - F_OUT), jnp.float32)], axis=1)


def _tc_b(agg1, y1, degp, W2, b1):
    return pl.pallas_call(
        _tcb_body,
        grid=(N // MB,),
        in_specs=[
            pl.BlockSpec((NC, MB, FH), lambda i: (0, i, 0)),
            pl.BlockSpec((NC, MB, FH), lambda i: (0, i, 0)),
            pl.BlockSpec((1, MB, 128), lambda i: (0, i, 0)),
            pl.BlockSpec((F_HID, F_OUT), lambda i: (0, 0)),
            pl.BlockSpec((1, F_HID), lambda i: (0, 0)),
        ],
        out_specs=pl.BlockSpec((MB, 128), lambda i: (i, 0)),
        out_shape=jax.ShapeDtypeStruct((NC * N, 128), jnp.float32),
    )(agg1, y1, degp, W2, b1)


def _tcc_body(a_ref, y_ref, dp_ref, b_ref, o_ref):
    dis = _dis_block(dp_ref)
    o_ref[...] = ((a_ref[0, :, :F_OUT] + y_ref[:, :F_OUT]) * dis
                  + b_ref[...])


def _tc_c(agg2, y2p, degp, b2):
    return pl.pallas_call(
        _tcc_body,
        grid=(N // MB,),
        in_specs=[
            pl.BlockSpec((1, MB, 128), lambda i: (0, i, 0)),
            pl.BlockSpec((MB, 128), lambda i: (i, 0)),
            pl.BlockSpec((1, MB, 128), lambda i: (0, i, 0)),
            pl.BlockSpec((1, F_OUT), lambda i: (0, 0)),
        ],
        out_specs=pl.BlockSpec((MB, F_OUT), lambda i: (i, 0)),
        out_shape=jax.ShapeDtypeStruct((N, F_OUT), jnp.float32),
    )(agg2, y2p, degp, b2)


def kernel(x, edge_index, W1, b1, W2, b2):
    src = edge_index[0].astype(jnp.int32)
    dst = edge_index[1].astype(jnp.int32)
    dst2 = jnp.concatenate([dst, dst])

    src_l1 = jnp.concatenate([src, src + N])
    ones_tab = jnp.broadcast_to(x[:1, :1] * 0.0 + 1.0, (NC * N, 128))
    degp = _edge_agg(ones_tab, src_l1, dst2).reshape(NC, NP, 128)

    y1 = _tc_a(x, W1, degp)
    agg1 = _edge_agg(y1, src_l1, dst2).reshape(NC, NP, 128)

    y2p = _tc_b(agg1, y1.reshape(NC, N, FH), degp, W2, b1.reshape(1, F_HID))
    src_l2 = jnp.concatenate([src, src])
    agg2 = _edge_agg(y2p, src_l2, dst2).reshape(NC, NP, 128)

    return _tc_c(agg2, y2p, degp, b2.reshape(1, F_OUT))
